# baseline (speedup 1.0000x reference)
import jax
import jax.numpy as jnp
from jax.experimental import pallas as pl
from jax.experimental.pallas import tpu as pltpu

N = 32768
DIM = 64
K = 1024
BN = 8192


def _assign_kernel(x_ref, c_ref, out_ref):
    x = x_ref[...]
    c = c_ref[...]
    c2 = jnp.sum(c * c, axis=1)
    cneg = c * (-2.0)
    dot = jax.lax.dot_general(
        cneg, x, (((1,), (1,)), ((), ())),
        preferred_element_type=jnp.float32)
    s = dot + c2[:, None]
    out_ref[...] = jnp.argmin(s, axis=0).astype(jnp.int32)


@jax.jit
def kernel(x, cluster_centers):
    return pl.pallas_call(
        _assign_kernel,
        grid=(N // BN,),
        in_specs=[
            pl.BlockSpec((BN, DIM), lambda i: (i, 0)),
            pl.BlockSpec((K, DIM), lambda i: (0, 0)),
        ],
        out_specs=pl.BlockSpec((BN,), lambda i: (i,)),
        out_shape=jax.ShapeDtypeStruct((N,), jnp.int32),
        compiler_params=pltpu.CompilerParams(
            dimension_semantics=("parallel",)),
    )(x, cluster_centers)
